# baseline (speedup 1.0000x reference)
import functools

import jax
import jax.numpy as jnp
from jax.experimental import pallas as pl

_E = 64
_K = 8
_SUB = 512
_NEG = -3.0e38


def _gate_block(h_ref, w_ref, ids_ref, wts_ref):
    h = h_ref[...]
    w = w_ref[...]
    logits = jax.lax.dot_general(
        w, h, (((1,), (1,)), ((), ())),
        preferred_element_type=jnp.float32)
    b = logits.shape[1]
    sub = jax.lax.broadcasted_iota(jnp.int32, (_E, _SUB), 0)
    for t in range(b // _SUB):
        tile = logits[:, t * _SUB:(t + 1) * _SUB]
        raw = jax.lax.bitcast_convert_type(tile, jnp.int32)
        low6 = jnp.where(raw < 0, sub, (_E - 1) - sub)
        key = jax.lax.bitcast_convert_type((raw & ~(_E - 1)) | low6,
                                           jnp.float32)
        ms = []
        for _ in range(_K):
            m = jnp.max(key, axis=0, keepdims=True)
            ms.append(m)
            key = jnp.where(key == m, _NEG, key)
        packed = jnp.concatenate(ms, axis=0)
        mi = jax.lax.bitcast_convert_type(packed, jnp.int32)
        low = mi & (_E - 1)
        ids_t = jnp.where(mi < 0, low, (_E - 1) - low)
        vals_t = jax.lax.bitcast_convert_type(mi & ~(_E - 1), jnp.float32)
        e = jnp.exp(vals_t - vals_t[:1, :])
        wts_t = e / jnp.sum(e, axis=0, keepdims=True)
        ids_ref[:, t * _SUB:(t + 1) * _SUB] = ids_t
        wts_ref[:, t * _SUB:(t + 1) * _SUB] = wts_t


@functools.partial(jax.jit, static_argnames=())
def kernel(h, W):
    b, s, d = h.shape
    n = b * s
    hf = h.reshape(n, d)
    blk = 8192
    grid = n // blk
    ids, wts = pl.pallas_call(
        _gate_block,
        grid=(grid,),
        in_specs=[
            pl.BlockSpec((blk, d), lambda i: (i, 0)),
            pl.BlockSpec((_E, d), lambda i: (0, 0)),
        ],
        out_specs=[
            pl.BlockSpec((_K, blk), lambda i: (0, i)),
            pl.BlockSpec((_K, blk), lambda i: (0, i)),
        ],
        out_shape=[
            jax.ShapeDtypeStruct((_K, n), jnp.int32),
            jax.ShapeDtypeStruct((_K, n), jnp.float32),
        ],
    )(hf, W)
    return ids.T, wts.T, jnp.float32(0.0)


# speedup vs baseline: 1.0102x; 1.0102x over previous
import functools

import jax
import jax.numpy as jnp
from jax.experimental import pallas as pl

_E = 64
_K = 8
_SUB = 512
_NEG = -3.0e38


def _gate_block(h_ref, w_ref, ids_ref, wts_ref):
    h = h_ref[...]
    w = w_ref[...]
    logits = jax.lax.dot_general(
        w, h, (((1,), (1,)), ((), ())),
        preferred_element_type=jnp.float32)
    b = logits.shape[1]
    sub = jax.lax.broadcasted_iota(jnp.int32, (_E, _SUB), 0)
    for t in range(b // _SUB):
        tile = logits[:, t * _SUB:(t + 1) * _SUB]
        raw = jax.lax.bitcast_convert_type(tile, jnp.int32)
        low6 = jnp.where(raw < 0, sub, (_E - 1) - sub)
        key = jax.lax.bitcast_convert_type((raw & ~(_E - 1)) | low6,
                                           jnp.float32)
        ms = []
        for _ in range(_K):
            m = jnp.max(key, axis=0, keepdims=True)
            ms.append(m)
            key = jnp.where(key == m, _NEG, key)
        packed = jnp.concatenate(ms, axis=0)
        mi = jax.lax.bitcast_convert_type(packed, jnp.int32)
        low = mi & (_E - 1)
        ids_t = jnp.where(mi < 0, low, (_E - 1) - low)
        vals_t = jax.lax.bitcast_convert_type(mi & ~(_E - 1), jnp.float32)
        e = jnp.exp(vals_t - vals_t[:1, :])
        wts_t = e / jnp.sum(e, axis=0, keepdims=True)
        ids_ref[:, t * _SUB:(t + 1) * _SUB] = ids_t
        wts_ref[:, t * _SUB:(t + 1) * _SUB] = wts_t


@functools.partial(jax.jit, static_argnames=())
def kernel(h, W):
    b, s, d = h.shape
    n = b * s
    hf = h.reshape(n, d)
    blk = 2048
    grid = n // blk
    ids, wts = pl.pallas_call(
        _gate_block,
        grid=(grid,),
        in_specs=[
            pl.BlockSpec((blk, d), lambda i: (i, 0)),
            pl.BlockSpec((_E, d), lambda i: (0, 0)),
        ],
        out_specs=[
            pl.BlockSpec((_K, blk), lambda i: (0, i)),
            pl.BlockSpec((_K, blk), lambda i: (0, i)),
        ],
        out_shape=[
            jax.ShapeDtypeStruct((_K, n), jnp.int32),
            jax.ShapeDtypeStruct((_K, n), jnp.float32),
        ],
    )(hf, W)
    return ids.T, wts.T, jnp.float32(0.0)


# speedup vs baseline: 1.0847x; 1.0738x over previous
import functools

import jax
import jax.numpy as jnp
from jax.experimental import pallas as pl

_E = 64
_K = 8
_SUB = 512
_NEG = -3.0e38


def _gate_block(h_ref, w_ref, ids_ref, wts_ref):
    h = h_ref[...]
    w = w_ref[...]
    logits = jax.lax.dot_general(
        w, h, (((1,), (1,)), ((), ())),
        preferred_element_type=jnp.float32)
    b = logits.shape[1]
    sub = jax.lax.broadcasted_iota(jnp.int32, (_E, _SUB), 0)
    for t in range(b // _SUB):
        tile = logits[:, t * _SUB:(t + 1) * _SUB]
        raw = jax.lax.bitcast_convert_type(tile, jnp.int32)
        low6 = jnp.where(raw < 0, sub, (_E - 1) - sub)
        key = jax.lax.bitcast_convert_type((raw & ~(_E - 1)) | low6,
                                           jnp.float32)
        ms = []
        for _ in range(_K):
            m = jnp.max(key, axis=0, keepdims=True)
            ms.append(m)
            key = jnp.where(key == m, _NEG, key)
        packed = jnp.concatenate(ms, axis=0)
        mi = jax.lax.bitcast_convert_type(packed, jnp.int32)
        low = mi & (_E - 1)
        ids_t = jnp.where(mi < 0, low, (_E - 1) - low)
        vals_t = jax.lax.bitcast_convert_type(mi & ~(_E - 1), jnp.float32)
        e = jnp.exp(vals_t - vals_t[:1, :])
        wts_t = e / jnp.sum(e, axis=0, keepdims=True)
        ids_ref[:, t * _SUB:(t + 1) * _SUB] = ids_t
        wts_ref[:, t * _SUB:(t + 1) * _SUB] = wts_t


@functools.partial(jax.jit, static_argnames=())
def kernel(h, W):
    b, s, d = h.shape
    n = b * s
    hf = h.reshape(n, d)
    blk = 4096
    grid = n // blk
    ids, wts = pl.pallas_call(
        _gate_block,
        grid=(grid,),
        in_specs=[
            pl.BlockSpec((blk, d), lambda i: (i, 0)),
            pl.BlockSpec((_E, d), lambda i: (0, 0)),
        ],
        out_specs=[
            pl.BlockSpec((_K, blk), lambda i: (0, i)),
            pl.BlockSpec((_K, blk), lambda i: (0, i)),
        ],
        out_shape=[
            jax.ShapeDtypeStruct((_K, n), jnp.int32),
            jax.ShapeDtypeStruct((_K, n), jnp.float32),
        ],
    )(hf, W)
    return ids.T, wts.T, jnp.float32(0.0)
